# baseline (speedup 1.0000x reference)
import functools

import jax
import jax.numpy as jnp
from jax import lax
from jax.experimental import pallas as pl
from jax.experimental.pallas import tpu as pltpu
from jax.experimental.pallas import tpu_sc as plsc

N = 100000
EMB = 112
DPE = 16
DH = 128
EPS = 1e-5
PACK = DH // DPE
N8 = N // PACK

_NC = 2
_NS = 16
_NW = _NC * _NS
_CHUNK = 128
_CPW = 25
_BPW = _CHUNK * _CPW


def _pe_body(pe_ref, bd_ref, bt_ref, gt_ref, bet_ref, out_ref):
    z = jnp.dot(pe_ref[...], bd_ref[...],
                preferred_element_type=jnp.float32) + bt_ref[...]
    s = jnp.sum(z, axis=0, keepdims=True)
    ss = jnp.sum(z * z, axis=0, keepdims=True)
    ri = lax.broadcasted_iota(jnp.int32, (DH, DH), 0) % DPE
    ci = lax.broadcasted_iota(jnp.int32, (DH, DH), 1) % DPE
    g = (ri == ci).astype(jnp.float32)
    stot = jnp.dot(s, g, preferred_element_type=jnp.float32)
    sstot = jnp.dot(ss, g, preferred_element_type=jnp.float32)
    mean = stot / N
    var = sstot / N - mean * mean
    scale = gt_ref[...] * lax.rsqrt(var + EPS)
    shift = bet_ref[...] - mean * scale
    out_ref[...] = z * scale + shift


def _pe_half(pe, W_pe, b_pe, gamma, beta):
    pe_r = pe.reshape(N8, DH)
    bd = jnp.kron(jnp.eye(PACK, dtype=jnp.float32), W_pe)
    bt = jnp.tile(b_pe, PACK)[None, :]
    gt = jnp.tile(gamma, PACK)[None, :]
    bet = jnp.tile(beta, PACK)[None, :]
    h2_r = pl.pallas_call(
        _pe_body,
        out_shape=jax.ShapeDtypeStruct((N8, DH), jnp.float32),
    )(pe_r, bd, bt, gt, bet)
    return h2_r.reshape(N, DPE)


def _sc_gather_concat(emb_table, feature, h2):
    mesh = plsc.VectorSubcoreMesh(core_axis_name="c", subcore_axis_name="s")

    @functools.partial(
        pl.kernel,
        mesh=mesh,
        out_type=jax.ShapeDtypeStruct((N, DH), jnp.float32),
        compiler_params=pltpu.CompilerParams(use_tc_tiling_on_sc=False),
        scratch_types=[
            pltpu.VMEM((1, _CHUNK), jnp.int32),
            pltpu.VMEM((_CHUNK, EMB), jnp.float32),
            pltpu.VMEM((_CHUNK, DPE), jnp.float32),
            pltpu.SemaphoreType.DMA,
        ],
    )
    def k(table_hbm, feat_hbm, h2_hbm, out_hbm, idx_v, rows_v, h2_v, sem):
        wid = lax.axis_index("s") * _NC + lax.axis_index("c")
        start = jnp.minimum(wid * _BPW, N - _BPW)

        def body(j, carry):
            r0 = start + j * _CHUNK
            pltpu.sync_copy(feat_hbm.at[pl.ds(r0, _CHUNK)], idx_v.at[0])
            pltpu.async_copy(table_hbm.at[idx_v.at[0]], rows_v, sem).wait()
            pltpu.sync_copy(h2_hbm.at[pl.ds(r0, _CHUNK)], h2_v)
            pltpu.sync_copy(rows_v, out_hbm.at[pl.ds(r0, _CHUNK), pl.ds(0, EMB)])
            pltpu.sync_copy(h2_v, out_hbm.at[pl.ds(r0, _CHUNK), pl.ds(EMB, DPE)])
            return carry

        lax.fori_loop(0, _CPW, body, 0)

    return k(emb_table, feature, h2)


def kernel(feature, pe, emb_table, W_pe, b_pe, gamma, beta):
    h2 = _pe_half(pe, W_pe, b_pe, gamma, beta)
    return _sc_gather_concat(emb_table, feature, h2)
